# Initial kernel scaffold; baseline (speedup 1.0000x reference)
import jax
import jax.numpy as jnp
from jax.experimental import pallas as pl


def kernel(x, edge_index, mask, labels, edge_weight, data, Ws, bs, Wp, bp, p_fill, p_reset, p_update, p_cell, p_final):
    raise NotImplementedError("write your pallas kernel here")


# speedup vs baseline: 9.6828x; 9.6828x over previous
import functools

import jax
import jax.numpy as jnp
from jax import lax
from jax.experimental import pallas as pl
from jax.experimental.pallas import tpu as pltpu
from jax.experimental.pallas import tpu_sc as plsc

_NC, _NS = 2, 16
_NW = _NC * _NS
_D = 16
_DW = 24
_BN = 2000
_BE = 2000
_CH = 1000

_f32 = jnp.float32


def _lrelu(v):
    return jnp.maximum(v, 0.2 * v)


def _pad_conv(p):
    dout = p["att"].shape[0]
    w = _D - dout
    return {
        "Wl": jnp.pad(p["Wl"], ((0, 0), (0, w))),
        "bl": jnp.pad(p["bl"], (0, w))[None, :],
        "Wr": jnp.pad(p["Wr"], ((0, 0), (0, w))),
        "br": jnp.pad(p["br"], (0, w))[None, :],
        "att": jnp.pad(p["att"], (0, w))[:, None],
        "b": jnp.pad(p["b"], (0, w))[None, :],
        "dout": dout,
    }


def _tc_tables(xcat, pp):
    n, din = xcat.shape
    grid = n // _BN

    def body(x_ref, wl_ref, bl_ref, wr_ref, br_ref, att_ref, src_ref, dst_ref):
        xb = x_ref[...]
        xl = jnp.dot(xb, wl_ref[...], preferred_element_type=_f32) + bl_ref[...]
        xr = jnp.dot(xb, wr_ref[...], preferred_element_type=_f32) + br_ref[...]
        h = _lrelu(xl + xr)
        el = jnp.dot(h, att_ref[...], preferred_element_type=_f32)
        src_ref[...] = xl
        dst_ref[...] = jnp.concatenate(
            [xr, el, jnp.zeros((_BN, _DW - _D - 1), _f32)], axis=1)

    full = lambda s: pl.BlockSpec(s, lambda i: (0, 0))
    return pl.pallas_call(
        body,
        grid=(grid,),
        in_specs=[
            pl.BlockSpec((_BN, din), lambda i: (i, 0)),
            full((din, _D)), full((1, _D)), full((din, _D)), full((1, _D)),
            full((_D, 1)),
        ],
        out_specs=[
            pl.BlockSpec((_BN, _D), lambda i: (i, 0)),
            pl.BlockSpec((_BN, _DW), lambda i: (i, 0)),
        ],
        out_shape=[
            jax.ShapeDtypeStruct((n, _D), _f32),
            jax.ShapeDtypeStruct((n, _DW), _f32),
        ],
    )(xcat, pp["Wl"], pp["bl"], pp["Wr"], pp["br"], pp["att"])


def _sc_gather(src_idx, dst_idx, src_tab, dst_tab):
    e_len = src_idx.shape[0]
    n = src_tab.shape[0]
    ew = e_len // _NW
    steps = ew // _CH
    assert ew % _CH == 0 and e_len % _NW == 0
    mesh = plsc.VectorSubcoreMesh(core_axis_name="c", subcore_axis_name="s")

    @functools.partial(
        pl.kernel,
        mesh=mesh,
        compiler_params=pltpu.CompilerParams(use_tc_tiling_on_sc=False),
        out_type=[
            jax.ShapeDtypeStruct((e_len, _D), _f32),
            jax.ShapeDtypeStruct((e_len, _DW), _f32),
        ],
        scratch_types=[
            pltpu.VMEM((_CH,), jnp.int32),
            pltpu.VMEM((_CH,), jnp.int32),
            pltpu.VMEM((_CH, _D), _f32),
            pltpu.VMEM((_CH, _DW), _f32),
            pltpu.SemaphoreType.DMA,
            pltpu.SemaphoreType.DMA,
        ],
    )
    def k(si_hbm, di_hbm, st_hbm, dt_hbm, gs_hbm, gd_hbm,
          si_v, di_v, rs_v, rd_v, sem_s, sem_d):
        wid = lax.axis_index("s") * _NC + lax.axis_index("c")
        base = wid * ew

        def step(i, carry):
            off = pl.multiple_of(base + i * _CH, 8)
            pltpu.sync_copy(si_hbm.at[pl.ds(off, _CH)], si_v)
            pltpu.sync_copy(di_hbm.at[pl.ds(off, _CH)], di_v)
            cs = pltpu.async_copy(st_hbm.at[si_v], rs_v, sem_s)
            cd = pltpu.async_copy(dt_hbm.at[di_v], rd_v, sem_d)
            cs.wait()
            cd.wait()
            pltpu.sync_copy(rs_v, gs_hbm.at[pl.ds(off, _CH)])
            pltpu.sync_copy(rd_v, gd_hbm.at[pl.ds(off, _CH)])
            return carry

        lax.fori_loop(0, steps, step, 0)

    return k(src_idx, dst_idx, src_tab, dst_tab)


def _tc_edge(gs, gd, att):
    e_len = gs.shape[0]
    grid = e_len // _BE

    def body(gs_ref, gd_ref, att_ref, p_ref):
        gsb = gs_ref[...]
        gdb = gd_ref[...]
        h = _lrelu(gsb + gdb[:, :_D])
        e = jnp.dot(h, att_ref[...], preferred_element_type=_f32)
        ez = jnp.exp(e - gdb[:, _D:_D + 1])
        p_ref[...] = jnp.concatenate(
            [ez * gsb, ez, jnp.zeros((_BE, _DW - _D - 1), _f32)], axis=1)

    return pl.pallas_call(
        body,
        grid=(grid,),
        in_specs=[
            pl.BlockSpec((_BE, _D), lambda i: (i, 0)),
            pl.BlockSpec((_BE, _DW), lambda i: (i, 0)),
            pl.BlockSpec((_D, 1), lambda i: (0, 0)),
        ],
        out_specs=pl.BlockSpec((_BE, _DW), lambda i: (i, 0)),
        out_shape=jax.ShapeDtypeStruct((e_len, _DW), _f32),
    )(gs, gd, att)


def _sc_scatter(dst_idx, p, zeros_nw):
    e_len = dst_idx.shape[0]
    n = zeros_nw.shape[0]
    ew = e_len // _NW
    steps = ew // _CH
    rpt = n // _NS
    mesh = plsc.VectorSubcoreMesh(core_axis_name="c", subcore_axis_name="s")

    @functools.partial(
        pl.kernel,
        mesh=mesh,
        compiler_params=pltpu.CompilerParams(use_tc_tiling_on_sc=False),
        out_type=jax.ShapeDtypeStruct((_NC, n, _DW), _f32),
        scratch_types=[
            pltpu.VMEM((_CH,), jnp.int32),
            pltpu.VMEM((_CH, _DW), _f32),
            pltpu.VMEM_SHARED((n, _DW), _f32),
        ],
    )
    def k(di_hbm, p_hbm, z_hbm, out_hbm, di_v, rows_v, acc):
        sid = lax.axis_index("s")
        cid = lax.axis_index("c")
        r0 = sid * rpt
        pltpu.sync_copy(z_hbm.at[pl.ds(r0, rpt)], acc.at[pl.ds(r0, rpt)])
        plsc.subcore_barrier()
        wid = sid * _NC + cid
        base = wid * ew

        def step(i, carry):
            off = pl.multiple_of(base + i * _CH, 8)
            pltpu.sync_copy(di_hbm.at[pl.ds(off, _CH)], di_v)
            pltpu.sync_copy(p_hbm.at[pl.ds(off, _CH)], rows_v)
            pltpu.sync_copy(rows_v, acc.at[di_v], add=True)
            return carry

        lax.fori_loop(0, steps, step, 0)
        plsc.subcore_barrier()
        pltpu.sync_copy(acc.at[pl.ds(r0, rpt)], out_hbm.at[cid, pl.ds(r0, rpt)])

    return k(dst_idx, p, zeros_nw)


def _combine(a_ref, xl_ref, b_ref):
    ab = a_ref[...]
    s = ab[0] + ab[1]
    num = s[:, :_D] + xl_ref[...]
    den = s[:, _D:_D + 1] + 1.0
    return num / den + b_ref[...]


def _acc_spec():
    return pl.BlockSpec((_NC, _BN, _DW), lambda i: (0, i, 0))


def _nspec(w):
    return pl.BlockSpec((_BN, w), lambda i: (i, 0))


def _full(s):
    return pl.BlockSpec(s, lambda i: (0, 0))


def _tc_fill(acc, xl, b, mask_f, mx, f_dim):
    n = xl.shape[0]

    def body(a_ref, xl_ref, b_ref, m_ref, mx_ref, o_ref):
        y2 = _combine(a_ref, xl_ref, b_ref)[:, :f_dim]
        o_ref[...] = jnp.where(m_ref[...] > 0.5, mx_ref[...], y2)

    return pl.pallas_call(
        body, grid=(n // _BN,),
        in_specs=[_acc_spec(), _nspec(_D), _full((1, _D)), _nspec(1), _nspec(f_dim)],
        out_specs=_nspec(f_dim),
        out_shape=jax.ShapeDtypeStruct((n, f_dim), _f32),
    )(acc, xl, b, mask_f, mx)


def _tc_gate_rh(acc, xl, b, hidden):
    n = xl.shape[0]

    def body(a_ref, xl_ref, b_ref, h_ref, o_ref):
        r = jax.nn.sigmoid(_combine(a_ref, xl_ref, b_ref))
        o_ref[...] = r * h_ref[...]

    return pl.pallas_call(
        body, grid=(n // _BN,),
        in_specs=[_acc_spec(), _nspec(_D), _full((1, _D)), _nspec(_D)],
        out_specs=_nspec(_D),
        out_shape=jax.ShapeDtypeStruct((n, _D), _f32),
    )(acc, xl, b, hidden)


def _tc_gate(acc, xl, b):
    n = xl.shape[0]

    def body(a_ref, xl_ref, b_ref, o_ref):
        o_ref[...] = jax.nn.sigmoid(_combine(a_ref, xl_ref, b_ref))

    return pl.pallas_call(
        body, grid=(n // _BN,),
        in_specs=[_acc_spec(), _nspec(_D), _full((1, _D))],
        out_specs=_nspec(_D),
        out_shape=jax.ShapeDtypeStruct((n, _D), _f32),
    )(acc, xl, b)


def _tc_cell(acc, xl, b, u, hidden):
    n = xl.shape[0]

    def body(a_ref, xl_ref, b_ref, u_ref, h_ref, o_ref):
        c = jnp.tanh(_combine(a_ref, xl_ref, b_ref))
        ub = u_ref[...]
        o_ref[...] = ub * h_ref[...] + (1.0 - ub) * c

    return pl.pallas_call(
        body, grid=(n // _BN,),
        in_specs=[_acc_spec(), _nspec(_D), _full((1, _D)), _nspec(_D), _nspec(_D)],
        out_specs=_nspec(_D),
        out_shape=jax.ShapeDtypeStruct((n, _D), _f32),
    )(acc, xl, b, u, hidden)


def _tc_final(acc, xl, b, static, Wp, bp):
    n = xl.shape[0]
    f_dim = Wp.shape[1]

    def body(a_ref, xl_ref, b_ref, st_ref, wp_ref, bp_ref, o_ref):
        s = jax.nn.relu(_combine(a_ref, xl_ref, b_ref))
        cat = jnp.concatenate([s, st_ref[...]], axis=1)
        o_ref[...] = jnp.dot(cat, wp_ref[...], preferred_element_type=_f32) + bp_ref[...]

    return pl.pallas_call(
        body, grid=(n // _BN,),
        in_specs=[_acc_spec(), _nspec(_D), _full((1, _D)), _nspec(_D),
                  _full((2 * _D, f_dim)), _full((1, f_dim))],
        out_specs=_nspec(f_dim),
        out_shape=jax.ShapeDtypeStruct((n, f_dim), _f32),
    )(acc, xl, b, static, Wp, bp[None, :])


def _tc_static(feat, Ws, bs):
    n, dfeat = feat.shape
    h_dim = Ws.shape[1]

    def body(f_ref, w_ref, b_ref, o_ref):
        o_ref[...] = jax.nn.relu(
            jnp.dot(f_ref[...], w_ref[...], preferred_element_type=_f32) + b_ref[...])

    return pl.pallas_call(
        body, grid=(n // _BN,),
        in_specs=[_nspec(dfeat), _full((dfeat, h_dim)), _full((1, h_dim))],
        out_specs=_nspec(h_dim),
        out_shape=jax.ShapeDtypeStruct((n, h_dim), _f32),
    )(feat, Ws, bs[None, :])


def _conv(xcat, src, dst, pp, zeros_nw):
    src_tab, dst_tab = _tc_tables(xcat, pp)
    gs, gd = _sc_gather(src, dst, src_tab, dst_tab)
    p = _tc_edge(gs, gd, pp["att"])
    acc = _sc_scatter(dst, p, zeros_nw)
    return acc, src_tab


def kernel(x, edge_index, mask, labels, edge_weight, data, Ws, bs, Wp, bp,
           p_fill, p_reset, p_update, p_cell, p_final):
    t_steps, n, f_dim = x.shape
    src, dst = edge_index[0], edge_index[1]
    m = mask[:, None]
    mf = m.astype(_f32)
    mx = jnp.where(m[None, :, :], x, 0.0)
    feat = jnp.concatenate([mf, 1.0 - mf, labels], axis=1)
    static = _tc_static(feat, Ws, bs)
    hidden = jnp.zeros((n, _D), _f32)
    zeros_nw = jnp.zeros((n, _DW), _f32)

    pf = _pad_conv(p_fill)
    pr = _pad_conv(p_reset)
    pu = _pad_conv(p_update)
    pc = _pad_conv(p_cell)
    pn = _pad_conv(p_final)

    preds = []
    for t in range(t_steps):
        acc, xl = _conv(jnp.concatenate([mx[t], static, hidden], axis=1),
                        src, dst, pf, zeros_nw)
        x2 = _tc_fill(acc, xl, pf["b"], mf, mx[t], f_dim)
        ft = jnp.concatenate([x2, static], axis=1)
        xg = jnp.concatenate([ft, hidden], axis=1)
        acc_r, xl_r = _conv(xg, src, dst, pr, zeros_nw)
        rh = _tc_gate_rh(acc_r, xl_r, pr["b"], hidden)
        acc_u, xl_u = _conv(xg, src, dst, pu, zeros_nw)
        u = _tc_gate(acc_u, xl_u, pu["b"])
        acc_c, xl_c = _conv(jnp.concatenate([ft, rh], axis=1),
                            src, dst, pc, zeros_nw)
        hidden = _tc_cell(acc_c, xl_c, pc["b"], u, hidden)
        acc_f, xl_f = _conv(jnp.concatenate([x2, static, hidden], axis=1),
                            src, dst, pn, zeros_nw)
        preds.append(_tc_final(acc_f, xl_f, pn["b"], static, Wp, bp))
    return jnp.stack(preds)


# speedup vs baseline: 35.4460x; 3.6607x over previous
import functools

import jax
import jax.numpy as jnp
from jax import lax
from jax.experimental import pallas as pl
from jax.experimental.pallas import tpu as pltpu
from jax.experimental.pallas import tpu_sc as plsc

_NC, _NS = 2, 16
_NW = _NC * _NS
_D = 16
_DW = 24
_BN = 2000
_CH = 512

_f32 = jnp.float32


def _lrelu(v):
    return jnp.maximum(v, 0.2 * v)


def _pad_conv(p):
    dout = p["att"].shape[0]
    w = _D - dout
    return {
        "Wl": jnp.pad(p["Wl"], ((0, 0), (0, w))),
        "bl": jnp.pad(p["bl"], (0, w))[None, :],
        "Wr": jnp.pad(p["Wr"], ((0, 0), (0, w))),
        "br": jnp.pad(p["br"], (0, w))[None, :],
        "att": jnp.pad(p["att"], (0, w))[:, None],
        "b": jnp.pad(p["b"], (0, w))[None, :],
    }


def _tc_tables(xcat, pp):
    n, din = xcat.shape
    grid = n // _BN

    def body(x_ref, wl_ref, bl_ref, wr_ref, br_ref, att_ref, src_ref, dst_ref):
        xb = x_ref[...]
        xl = jnp.dot(xb, wl_ref[...], preferred_element_type=_f32) + bl_ref[...]
        xr = jnp.dot(xb, wr_ref[...], preferred_element_type=_f32) + br_ref[...]
        h = _lrelu(xl + xr)
        el = jnp.dot(h, att_ref[...], preferred_element_type=_f32)
        src_ref[...] = xl
        dst_ref[...] = jnp.concatenate(
            [xr, el, jnp.zeros((_BN, _DW - _D - 1), _f32)], axis=1)

    full = lambda s: pl.BlockSpec(s, lambda i: (0, 0))
    return pl.pallas_call(
        body,
        grid=(grid,),
        in_specs=[
            pl.BlockSpec((_BN, din), lambda i: (i, 0)),
            full((din, _D)), full((1, _D)), full((din, _D)), full((1, _D)),
            full((_D, 1)),
        ],
        out_specs=[
            pl.BlockSpec((_BN, _D), lambda i: (i, 0)),
            pl.BlockSpec((_BN, _DW), lambda i: (i, 0)),
        ],
        out_shape=[
            jax.ShapeDtypeStruct((n, _D), _f32),
            jax.ShapeDtypeStruct((n, _DW), _f32),
        ],
    )(xcat, pp["Wl"], pp["bl"], pp["Wr"], pp["br"], pp["att"])


def _sc_conv(src_p, dst_p, src_tab, dst_tab, att_col, zeros_nw, e_real):
    e_pad = src_p.shape[0]
    n = src_tab.shape[0]
    ew = e_pad // _NW
    steps = ew // _CH
    groups = _CH // 16
    rpt = n // _NS
    assert ew % _CH == 0 and _CH % 16 == 0 and n % _NS == 0
    mesh = plsc.VectorSubcoreMesh(core_axis_name="c", subcore_axis_name="s")

    @functools.partial(
        pl.kernel,
        mesh=mesh,
        compiler_params=pltpu.CompilerParams(
            use_tc_tiling_on_sc=False, needs_layout_passes=False),
        out_type=jax.ShapeDtypeStruct((_NC, n, _DW), _f32),
        scratch_types=[
            pltpu.VMEM((16, 16), _f32),
            pltpu.VMEM((_CH,), jnp.int32),
            pltpu.VMEM((_CH,), jnp.int32),
            pltpu.VMEM((_CH, _D), _f32),
            pltpu.VMEM((_CH, _DW), _f32),
            pltpu.VMEM((_CH, _DW), _f32),
            pltpu.VMEM_SHARED((n, _DW), _f32),
            pltpu.SemaphoreType.DMA,
            pltpu.SemaphoreType.DMA,
        ],
    )
    def k(si_hbm, di_hbm, st_hbm, dt_hbm, att_hbm, z_hbm, out_hbm,
          att_v, si_v, di_v, rs_v, rd_v, p_v, acc, sem_s, sem_d):
        sid = lax.axis_index("s")
        cid = lax.axis_index("c")
        r0 = sid * rpt
        pltpu.sync_copy(z_hbm.at[pl.ds(r0, rpt)], acc.at[pl.ds(r0, rpt)])
        pltpu.sync_copy(att_hbm, att_v)
        plsc.subcore_barrier()
        wid = sid * _NC + cid
        base = wid * ew

        def chunk(i, carry):
            off = pl.multiple_of(base + i * _CH, 8)
            pltpu.sync_copy(si_hbm.at[pl.ds(off, _CH)], si_v)
            pltpu.sync_copy(di_hbm.at[pl.ds(off, _CH)], di_v)
            cs = pltpu.async_copy(st_hbm.at[si_v], rs_v, sem_s)
            cd = pltpu.async_copy(dt_hbm.at[di_v], rd_v, sem_d)
            cs.wait()
            cd.wait()

            def group(j, carry2):
                lanes = lax.iota(jnp.int32, 16) + j * 16
                eacc = jnp.zeros((16,), _f32)
                cols = []
                for kf in range(_D):
                    kidx = jnp.full((16,), kf, jnp.int32)
                    a = plsc.load_gather(rs_v, [lanes, kidx])
                    b = plsc.load_gather(rd_v, [lanes, kidx])
                    cols.append(a)
                    eacc = eacc + att_v[kf] * _lrelu(a + b)
                el = plsc.load_gather(rd_v, [lanes, jnp.full((16,), _D, jnp.int32)])
                pos = off + j * 16 + lanes
                live = jnp.where(pos < e_real, 1.0, 0.0).astype(_f32)
                y = (eacc - el) * 1.4426950408889634
                y = jnp.minimum(jnp.maximum(y, -120.0), 120.0)
                n_i = y.astype(jnp.int32)
                r = (y - n_i.astype(_f32)) * 0.6931471805599453
                p = 1.0 + r * (1.0 + r * (0.5 + r * (
                    0.16666666666666666 + r * (0.041666666666666664 + r * (
                        0.008333333333333333 + r * (
                            0.001388888888888889 + r * 0.0001984126984126984))))))
                scale = plsc.bitcast((n_i + 127) << 23, _f32)
                ez = p * scale * live
                for kf in range(_D):
                    plsc.store_scatter(
                        p_v, [lanes, jnp.full((16,), kf, jnp.int32)],
                        cols[kf] * ez)
                plsc.store_scatter(
                    p_v, [lanes, jnp.full((16,), _D, jnp.int32)], ez)
                return carry2

            lax.fori_loop(0, groups, group, 0)
            pltpu.sync_copy(p_v, acc.at[di_v], add=True)
            return carry

        lax.fori_loop(0, steps, chunk, 0)
        plsc.subcore_barrier()
        pltpu.sync_copy(acc.at[pl.ds(r0, rpt)], out_hbm.at[cid, pl.ds(r0, rpt)])

    return k(src_p, dst_p, src_tab, dst_tab, att_col, zeros_nw)


def _combine(a_ref, xl_ref, b_ref):
    ab = a_ref[...]
    s = ab[0] + ab[1]
    num = s[:, :_D] + xl_ref[...]
    den = s[:, _D:_D + 1] + 1.0
    return num / den + b_ref[...]


def _acc_spec():
    return pl.BlockSpec((_NC, _BN, _DW), lambda i: (0, i, 0))


def _nspec(w):
    return pl.BlockSpec((_BN, w), lambda i: (i, 0))


def _full(s):
    return pl.BlockSpec(s, lambda i: (0, 0))


def _tc_fill(acc, xl, b, mask_f, mx, f_dim):
    n = xl.shape[0]

    def body(a_ref, xl_ref, b_ref, m_ref, mx_ref, o_ref):
        y2 = _combine(a_ref, xl_ref, b_ref)[:, :f_dim]
        o_ref[...] = jnp.where(m_ref[...] > 0.5, mx_ref[...], y2)

    return pl.pallas_call(
        body, grid=(n // _BN,),
        in_specs=[_acc_spec(), _nspec(_D), _full((1, _D)), _nspec(1), _nspec(f_dim)],
        out_specs=_nspec(f_dim),
        out_shape=jax.ShapeDtypeStruct((n, f_dim), _f32),
    )(acc, xl, b, mask_f, mx)


def _tc_gate_rh(acc, xl, b, hidden):
    n = xl.shape[0]

    def body(a_ref, xl_ref, b_ref, h_ref, o_ref):
        r = jax.nn.sigmoid(_combine(a_ref, xl_ref, b_ref))
        o_ref[...] = r * h_ref[...]

    return pl.pallas_call(
        body, grid=(n // _BN,),
        in_specs=[_acc_spec(), _nspec(_D), _full((1, _D)), _nspec(_D)],
        out_specs=_nspec(_D),
        out_shape=jax.ShapeDtypeStruct((n, _D), _f32),
    )(acc, xl, b, hidden)


def _tc_gate(acc, xl, b):
    n = xl.shape[0]

    def body(a_ref, xl_ref, b_ref, o_ref):
        o_ref[...] = jax.nn.sigmoid(_combine(a_ref, xl_ref, b_ref))

    return pl.pallas_call(
        body, grid=(n // _BN,),
        in_specs=[_acc_spec(), _nspec(_D), _full((1, _D))],
        out_specs=_nspec(_D),
        out_shape=jax.ShapeDtypeStruct((n, _D), _f32),
    )(acc, xl, b)


def _tc_cell(acc, xl, b, u, hidden):
    n = xl.shape[0]

    def body(a_ref, xl_ref, b_ref, u_ref, h_ref, o_ref):
        c = jnp.tanh(_combine(a_ref, xl_ref, b_ref))
        ub = u_ref[...]
        o_ref[...] = ub * h_ref[...] + (1.0 - ub) * c

    return pl.pallas_call(
        body, grid=(n // _BN,),
        in_specs=[_acc_spec(), _nspec(_D), _full((1, _D)), _nspec(_D), _nspec(_D)],
        out_specs=_nspec(_D),
        out_shape=jax.ShapeDtypeStruct((n, _D), _f32),
    )(acc, xl, b, u, hidden)


def _tc_final(acc, xl, b, static, Wp, bp):
    n = xl.shape[0]
    f_dim = Wp.shape[1]

    def body(a_ref, xl_ref, b_ref, st_ref, wp_ref, bp_ref, o_ref):
        s = jax.nn.relu(_combine(a_ref, xl_ref, b_ref))
        cat = jnp.concatenate([s, st_ref[...]], axis=1)
        o_ref[...] = jnp.dot(cat, wp_ref[...], preferred_element_type=_f32) + bp_ref[...]

    return pl.pallas_call(
        body, grid=(n // _BN,),
        in_specs=[_acc_spec(), _nspec(_D), _full((1, _D)), _nspec(_D),
                  _full((2 * _D, f_dim)), _full((1, f_dim))],
        out_specs=_nspec(f_dim),
        out_shape=jax.ShapeDtypeStruct((n, f_dim), _f32),
    )(acc, xl, b, static, Wp, bp[None, :])


def _tc_static(feat, Ws, bs):
    n, dfeat = feat.shape
    h_dim = Ws.shape[1]

    def body(f_ref, w_ref, b_ref, o_ref):
        o_ref[...] = jax.nn.relu(
            jnp.dot(f_ref[...], w_ref[...], preferred_element_type=_f32) + b_ref[...])

    return pl.pallas_call(
        body, grid=(n // _BN,),
        in_specs=[_nspec(dfeat), _full((dfeat, h_dim)), _full((1, h_dim))],
        out_specs=_nspec(h_dim),
        out_shape=jax.ShapeDtypeStruct((n, h_dim), _f32),
    )(feat, Ws, bs[None, :])


def _conv(xcat, src_p, dst_p, pp, zeros_nw, e_real):
    src_tab, dst_tab = _tc_tables(xcat, pp)
    att_mat = jnp.broadcast_to(pp["att"], (16, 16))
    acc = _sc_conv(src_p, dst_p, src_tab, dst_tab, att_mat,
                   zeros_nw, e_real)
    return acc, src_tab


def kernel(x, edge_index, mask, labels, edge_weight, data, Ws, bs, Wp, bp,
           p_fill, p_reset, p_update, p_cell, p_final):
    t_steps, n, f_dim = x.shape
    e_real = edge_index.shape[1]
    e_pad = ((e_real + _NW * _CH - 1) // (_NW * _CH)) * (_NW * _CH)
    padn = e_pad - e_real
    src_p = jnp.concatenate([edge_index[0], jnp.zeros((padn,), jnp.int32)])
    dst_p = jnp.concatenate([edge_index[1], jnp.zeros((padn,), jnp.int32)])
    m = mask[:, None]
    mf = m.astype(_f32)
    mx = jnp.where(m[None, :, :], x, 0.0)
    feat = jnp.concatenate([mf, 1.0 - mf, labels], axis=1)
    static = _tc_static(feat, Ws, bs)
    hidden = jnp.zeros((n, _D), _f32)
    zeros_nw = jnp.zeros((n, _DW), _f32)

    pf = _pad_conv(p_fill)
    pr = _pad_conv(p_reset)
    pu = _pad_conv(p_update)
    pc = _pad_conv(p_cell)
    pn = _pad_conv(p_final)

    preds = []
    for t in range(t_steps):
        acc, xl = _conv(jnp.concatenate([mx[t], static, hidden], axis=1),
                        src_p, dst_p, pf, zeros_nw, e_real)
        x2 = _tc_fill(acc, xl, pf["b"], mf, mx[t], f_dim)
        ft = jnp.concatenate([x2, static], axis=1)
        xg = jnp.concatenate([ft, hidden], axis=1)
        acc_r, xl_r = _conv(xg, src_p, dst_p, pr, zeros_nw, e_real)
        rh = _tc_gate_rh(acc_r, xl_r, pr["b"], hidden)
        acc_u, xl_u = _conv(xg, src_p, dst_p, pu, zeros_nw, e_real)
        u = _tc_gate(acc_u, xl_u, pu["b"])
        acc_c, xl_c = _conv(jnp.concatenate([ft, rh], axis=1),
                            src_p, dst_p, pc, zeros_nw, e_real)
        hidden = _tc_cell(acc_c, xl_c, pc["b"], u, hidden)
        acc_f, xl_f = _conv(jnp.concatenate([x2, static, hidden], axis=1),
                            src_p, dst_p, pn, zeros_nw, e_real)
        preds.append(_tc_final(acc_f, xl_f, pn["b"], static, Wp, bp))
    return jnp.stack(preds)


# speedup vs baseline: 38.1234x; 1.0755x over previous
import functools

import jax
import jax.numpy as jnp
from jax import lax
from jax.experimental import pallas as pl
from jax.experimental.pallas import tpu as pltpu
from jax.experimental.pallas import tpu_sc as plsc

_NC, _NS = 2, 16
_NW = _NC * _NS
_D = 16
_DW = 24
_BN = 2000
_CH = 448

_f32 = jnp.float32


def _lrelu(v):
    return jnp.maximum(v, 0.2 * v)


def _pad_conv(p):
    dout = p["att"].shape[0]
    w = _D - dout
    return {
        "Wl": jnp.pad(p["Wl"], ((0, 0), (0, w))),
        "bl": jnp.pad(p["bl"], (0, w))[None, :],
        "Wr": jnp.pad(p["Wr"], ((0, 0), (0, w))),
        "br": jnp.pad(p["br"], (0, w))[None, :],
        "att": jnp.pad(p["att"], (0, w))[:, None],
        "b": jnp.pad(p["b"], (0, w))[None, :],
    }


def _tc_tables(xcat, pp):
    n, din = xcat.shape
    grid = n // _BN

    def body(x_ref, wl_ref, bl_ref, wr_ref, br_ref, att_ref, src_ref, dst_ref):
        xb = x_ref[...]
        xl = jnp.dot(xb, wl_ref[...], preferred_element_type=_f32) + bl_ref[...]
        xr = jnp.dot(xb, wr_ref[...], preferred_element_type=_f32) + br_ref[...]
        h = _lrelu(xl + xr)
        el = jnp.dot(h, att_ref[...], preferred_element_type=_f32)
        src_ref[...] = xl
        dst_ref[...] = jnp.concatenate(
            [xr, el, jnp.zeros((_BN, _DW - _D - 1), _f32)], axis=1)

    full = lambda s: pl.BlockSpec(s, lambda i: (0, 0))
    return pl.pallas_call(
        body,
        grid=(grid,),
        in_specs=[
            pl.BlockSpec((_BN, din), lambda i: (i, 0)),
            full((din, _D)), full((1, _D)), full((din, _D)), full((1, _D)),
            full((_D, 1)),
        ],
        out_specs=[
            pl.BlockSpec((_BN, _D), lambda i: (i, 0)),
            pl.BlockSpec((_BN, _DW), lambda i: (i, 0)),
        ],
        out_shape=[
            jax.ShapeDtypeStruct((n, _D), _f32),
            jax.ShapeDtypeStruct((n, _DW), _f32),
        ],
    )(xcat, pp["Wl"], pp["bl"], pp["Wr"], pp["br"], pp["att"])


def _sc_conv(src_p, dst_p, src_tab, dst_tab, att_col, zeros_nw, e_real):
    e_pad = src_p.shape[0]
    n = src_tab.shape[0]
    ew = e_pad // _NW
    steps = ew // _CH
    groups = _CH // 16
    rpt = n // _NS
    assert ew % _CH == 0 and steps % 2 == 0 and _CH % 16 == 0 and n % _NS == 0
    mesh = plsc.VectorSubcoreMesh(core_axis_name="c", subcore_axis_name="s")

    @functools.partial(
        pl.kernel,
        mesh=mesh,
        compiler_params=pltpu.CompilerParams(
            use_tc_tiling_on_sc=False, needs_layout_passes=False),
        out_type=jax.ShapeDtypeStruct((_NC, n, _DW), _f32),
        scratch_types=[
            pltpu.VMEM((16, 16), _f32),
            pltpu.VMEM((_CH,), jnp.int32),
            pltpu.VMEM((_CH,), jnp.int32),
            pltpu.VMEM((_CH,), jnp.int32),
            pltpu.VMEM((_CH,), jnp.int32),
            pltpu.VMEM((_CH, _D), _f32),
            pltpu.VMEM((_CH, _D), _f32),
            pltpu.VMEM((_CH, _DW), _f32),
            pltpu.VMEM((_CH, _DW), _f32),
            pltpu.VMEM((_CH, _DW), _f32),
            pltpu.VMEM_SHARED((n, _DW), _f32),
            pltpu.SemaphoreType.DMA,
            pltpu.SemaphoreType.DMA,
            pltpu.SemaphoreType.DMA,
            pltpu.SemaphoreType.DMA,
        ],
    )
    def k(si_hbm, di_hbm, st_hbm, dt_hbm, att_hbm, z_hbm, out_hbm,
          att_v, si0, si1, di0, di1, rs0, rs1, rd0, rd1, p_v, acc,
          sem_i0, sem_i1, sem_g0, sem_g1):
        bufs = ((si0, di0, rs0, rd0, sem_i0, sem_g0),
                (si1, di1, rs1, rd1, sem_i1, sem_g1))
        sid = lax.axis_index("s")
        cid = lax.axis_index("c")
        r0 = sid * rpt
        pltpu.sync_copy(z_hbm.at[pl.ds(r0, rpt)], acc.at[pl.ds(r0, rpt)])
        pltpu.sync_copy(att_hbm, att_v)
        plsc.subcore_barrier()
        wid = sid * _NC + cid
        base = wid * ew

        def compute(off, rs_c, rd_c):
            def group(j, carry2):
                lanes = lax.iota(jnp.int32, 16) + j * 16
                eacc = jnp.zeros((16,), _f32)
                cols = []
                for kf in range(_D):
                    kidx = jnp.full((16,), kf, jnp.int32)
                    a = plsc.load_gather(rs_c, [lanes, kidx])
                    b = plsc.load_gather(rd_c, [lanes, kidx])
                    cols.append(a)
                    eacc = eacc + att_v[kf] * _lrelu(a + b)
                el = plsc.load_gather(rd_c, [lanes, jnp.full((16,), _D, jnp.int32)])
                pos = off + j * 16 + lanes
                live = jnp.where(pos < e_real, 1.0, 0.0).astype(_f32)
                y = (eacc - el) * 1.4426950408889634
                y = jnp.minimum(jnp.maximum(y, -120.0), 120.0)
                n_i = y.astype(jnp.int32)
                r = (y - n_i.astype(_f32)) * 0.6931471805599453
                p = 1.0 + r * (1.0 + r * (0.5 + r * (
                    0.16666666666666666 + r * (0.041666666666666664 + r * (
                        0.008333333333333333 + r * (
                            0.001388888888888889 + r * 0.0001984126984126984))))))
                scale = plsc.bitcast((n_i + 127) << 23, _f32)
                ez = p * scale * live
                for kf in range(_D):
                    plsc.store_scatter(
                        p_v, [lanes, jnp.full((16,), kf, jnp.int32)],
                        cols[kf] * ez)
                plsc.store_scatter(
                    p_v, [lanes, jnp.full((16,), _D, jnp.int32)], ez)
                return carry2

            lax.fori_loop(0, groups, group, 0)

        off0 = pl.multiple_of(base, 8)
        pltpu.sync_copy(si_hbm.at[pl.ds(off0, _CH)], si0)
        pltpu.sync_copy(di_hbm.at[pl.ds(off0, _CH)], di0)
        pltpu.async_copy(st_hbm.at[si0], rs0, sem_g0)
        pltpu.async_copy(dt_hbm.at[di0], rd0, sem_g0)

        def pair(ip, carry):
            for b in (0, 1):
                i = ip * 2 + b
                si_c, di_c, rs_c, rd_c, _, sem_gc = bufs[b]
                si_n, di_n, rs_n, rd_n, sem_in, sem_gn = bufs[1 - b]
                i_n = jnp.minimum(i + 1, steps - 1)
                off_n = pl.multiple_of(base + i_n * _CH, 8)
                pltpu.async_copy(si_hbm.at[pl.ds(off_n, _CH)], si_n, sem_in)
                pltpu.async_copy(di_hbm.at[pl.ds(off_n, _CH)], di_n, sem_in)
                pltpu.make_async_copy(st_hbm.at[si_c], rs_c, sem_gc).wait()
                pltpu.make_async_copy(dt_hbm.at[di_c], rd_c, sem_gc).wait()
                compute(pl.multiple_of(base + i * _CH, 8), rs_c, rd_c)
                pltpu.make_async_copy(
                    si_hbm.at[pl.ds(off_n, _CH)], si_n, sem_in).wait()
                pltpu.make_async_copy(
                    di_hbm.at[pl.ds(off_n, _CH)], di_n, sem_in).wait()
                pltpu.async_copy(st_hbm.at[si_n], rs_n, sem_gn)
                pltpu.async_copy(dt_hbm.at[di_n], rd_n, sem_gn)
                pltpu.sync_copy(p_v, acc.at[di_c], add=True)
            return carry

        lax.fori_loop(0, steps // 2, pair, 0)
        pltpu.make_async_copy(st_hbm.at[si0], rs0, sem_g0).wait()
        pltpu.make_async_copy(dt_hbm.at[di0], rd0, sem_g0).wait()
        plsc.subcore_barrier()
        pltpu.sync_copy(acc.at[pl.ds(r0, rpt)], out_hbm.at[cid, pl.ds(r0, rpt)])

    return k(src_p, dst_p, src_tab, dst_tab, att_col, zeros_nw)


def _combine(a_ref, xl_ref, b_ref):
    ab = a_ref[...]
    s = ab[0] + ab[1]
    num = s[:, :_D] + xl_ref[...]
    den = s[:, _D:_D + 1] + 1.0
    return num / den + b_ref[...]


def _acc_spec():
    return pl.BlockSpec((_NC, _BN, _DW), lambda i: (0, i, 0))


def _nspec(w):
    return pl.BlockSpec((_BN, w), lambda i: (i, 0))


def _full(s):
    return pl.BlockSpec(s, lambda i: (0, 0))


def _tc_fill(acc, xl, b, mask_f, mx, f_dim):
    n = xl.shape[0]

    def body(a_ref, xl_ref, b_ref, m_ref, mx_ref, o_ref):
        y2 = _combine(a_ref, xl_ref, b_ref)[:, :f_dim]
        o_ref[...] = jnp.where(m_ref[...] > 0.5, mx_ref[...], y2)

    return pl.pallas_call(
        body, grid=(n // _BN,),
        in_specs=[_acc_spec(), _nspec(_D), _full((1, _D)), _nspec(1), _nspec(f_dim)],
        out_specs=_nspec(f_dim),
        out_shape=jax.ShapeDtypeStruct((n, f_dim), _f32),
    )(acc, xl, b, mask_f, mx)


def _tc_gate_rh(acc, xl, b, hidden):
    n = xl.shape[0]

    def body(a_ref, xl_ref, b_ref, h_ref, o_ref):
        r = jax.nn.sigmoid(_combine(a_ref, xl_ref, b_ref))
        o_ref[...] = r * h_ref[...]

    return pl.pallas_call(
        body, grid=(n // _BN,),
        in_specs=[_acc_spec(), _nspec(_D), _full((1, _D)), _nspec(_D)],
        out_specs=_nspec(_D),
        out_shape=jax.ShapeDtypeStruct((n, _D), _f32),
    )(acc, xl, b, hidden)


def _tc_gate(acc, xl, b):
    n = xl.shape[0]

    def body(a_ref, xl_ref, b_ref, o_ref):
        o_ref[...] = jax.nn.sigmoid(_combine(a_ref, xl_ref, b_ref))

    return pl.pallas_call(
        body, grid=(n // _BN,),
        in_specs=[_acc_spec(), _nspec(_D), _full((1, _D))],
        out_specs=_nspec(_D),
        out_shape=jax.ShapeDtypeStruct((n, _D), _f32),
    )(acc, xl, b)


def _tc_cell(acc, xl, b, u, hidden):
    n = xl.shape[0]

    def body(a_ref, xl_ref, b_ref, u_ref, h_ref, o_ref):
        c = jnp.tanh(_combine(a_ref, xl_ref, b_ref))
        ub = u_ref[...]
        o_ref[...] = ub * h_ref[...] + (1.0 - ub) * c

    return pl.pallas_call(
        body, grid=(n // _BN,),
        in_specs=[_acc_spec(), _nspec(_D), _full((1, _D)), _nspec(_D), _nspec(_D)],
        out_specs=_nspec(_D),
        out_shape=jax.ShapeDtypeStruct((n, _D), _f32),
    )(acc, xl, b, u, hidden)


def _tc_final(acc, xl, b, static, Wp, bp):
    n = xl.shape[0]
    f_dim = Wp.shape[1]

    def body(a_ref, xl_ref, b_ref, st_ref, wp_ref, bp_ref, o_ref):
        s = jax.nn.relu(_combine(a_ref, xl_ref, b_ref))
        cat = jnp.concatenate([s, st_ref[...]], axis=1)
        o_ref[...] = jnp.dot(cat, wp_ref[...], preferred_element_type=_f32) + bp_ref[...]

    return pl.pallas_call(
        body, grid=(n // _BN,),
        in_specs=[_acc_spec(), _nspec(_D), _full((1, _D)), _nspec(_D),
                  _full((2 * _D, f_dim)), _full((1, f_dim))],
        out_specs=_nspec(f_dim),
        out_shape=jax.ShapeDtypeStruct((n, f_dim), _f32),
    )(acc, xl, b, static, Wp, bp[None, :])


def _tc_static(feat, Ws, bs):
    n, dfeat = feat.shape
    h_dim = Ws.shape[1]

    def body(f_ref, w_ref, b_ref, o_ref):
        o_ref[...] = jax.nn.relu(
            jnp.dot(f_ref[...], w_ref[...], preferred_element_type=_f32) + b_ref[...])

    return pl.pallas_call(
        body, grid=(n // _BN,),
        in_specs=[_nspec(dfeat), _full((dfeat, h_dim)), _full((1, h_dim))],
        out_specs=_nspec(h_dim),
        out_shape=jax.ShapeDtypeStruct((n, h_dim), _f32),
    )(feat, Ws, bs[None, :])


def _conv(xcat, src_p, dst_p, pp, zeros_nw, e_real):
    src_tab, dst_tab = _tc_tables(xcat, pp)
    att_mat = jnp.broadcast_to(pp["att"], (16, 16))
    acc = _sc_conv(src_p, dst_p, src_tab, dst_tab, att_mat,
                   zeros_nw, e_real)
    return acc, src_tab


def kernel(x, edge_index, mask, labels, edge_weight, data, Ws, bs, Wp, bp,
           p_fill, p_reset, p_update, p_cell, p_final):
    t_steps, n, f_dim = x.shape
    e_real = edge_index.shape[1]
    unit = _NW * _CH * 2
    e_pad = ((e_real + unit - 1) // unit) * unit
    padn = e_pad - e_real
    src_p = jnp.concatenate([edge_index[0], jnp.zeros((padn,), jnp.int32)])
    dst_p = jnp.concatenate([edge_index[1], jnp.zeros((padn,), jnp.int32)])
    m = mask[:, None]
    mf = m.astype(_f32)
    mx = jnp.where(m[None, :, :], x, 0.0)
    feat = jnp.concatenate([mf, 1.0 - mf, labels], axis=1)
    static = _tc_static(feat, Ws, bs)
    hidden = jnp.zeros((n, _D), _f32)
    zeros_nw = jnp.zeros((n, _DW), _f32)

    pf = _pad_conv(p_fill)
    pr = _pad_conv(p_reset)
    pu = _pad_conv(p_update)
    pc = _pad_conv(p_cell)
    pn = _pad_conv(p_final)

    preds = []
    for t in range(t_steps):
        acc, xl = _conv(jnp.concatenate([mx[t], static, hidden], axis=1),
                        src_p, dst_p, pf, zeros_nw, e_real)
        x2 = _tc_fill(acc, xl, pf["b"], mf, mx[t], f_dim)
        ft = jnp.concatenate([x2, static], axis=1)
        xg = jnp.concatenate([ft, hidden], axis=1)
        acc_r, xl_r = _conv(xg, src_p, dst_p, pr, zeros_nw, e_real)
        rh = _tc_gate_rh(acc_r, xl_r, pr["b"], hidden)
        acc_u, xl_u = _conv(xg, src_p, dst_p, pu, zeros_nw, e_real)
        u = _tc_gate(acc_u, xl_u, pu["b"])
        acc_c, xl_c = _conv(jnp.concatenate([ft, rh], axis=1),
                            src_p, dst_p, pc, zeros_nw, e_real)
        hidden = _tc_cell(acc_c, xl_c, pc["b"], u, hidden)
        acc_f, xl_f = _conv(jnp.concatenate([x2, static, hidden], axis=1),
                            src_p, dst_p, pn, zeros_nw, e_real)
        preds.append(_tc_final(acc_f, xl_f, pn["b"], static, Wp, bp))
    return jnp.stack(preds)
